# baseline (speedup 1.0000x reference)
import functools

import jax
import jax.numpy as jnp
from jax import lax
from jax.experimental import pallas as pl
from jax.experimental.pallas import tpu as pltpu
from jax.experimental.pallas import tpu_sc as plsc

_NC = 2
_NS = 16
_NW = _NC * _NS

_BIG = 2**31 - 1


def _sc_mesh():
    return plsc.VectorSubcoreMesh(
        core_axis_name="c", subcore_axis_name="s", num_cores=_NC,
        num_subcores=_NS)


def _worker_id():
    return lax.axis_index("s") * _NC + lax.axis_index("c")


def _sc_scatter_prio(set_indices, n_pad):
    B = set_indices.shape[0]
    PW = n_pad // _NW
    CH = 10000
    NCH = B // CH

    @functools.partial(
        pl.kernel,
        out_type=jax.ShapeDtypeStruct((n_pad,), jnp.int32),
        mesh=_sc_mesh(),
        compiler_params=pltpu.CompilerParams(needs_layout_passes=False),
        scratch_types=[
            pltpu.VMEM((PW,), jnp.int32),
            pltpu.VMEM((CH,), jnp.int32),
        ],
    )
    def scatter_k(idx_hbm, out_hbm, prio_v, idx_v):
        wid = _worker_id()
        lo = wid * PW

        def init(i, _):
            prio_v[pl.ds(i * 16, 16)] = jnp.full((16,), _BIG, jnp.int32)
            return _

        lax.fori_loop(0, PW // 16, init, None)

        iota = lax.iota(jnp.int32, 16)

        def chunk(t, _):
            c = NCH - 1 - t
            pltpu.sync_copy(idx_hbm.at[pl.ds(c * CH, CH)], idx_v)

            def vec(j, _):
                jj = (CH // 16) - 1 - j
                v = idx_v[pl.ds(jj * 16, 16)]
                pos = (c * CH + jj * 16) + iota
                rel = v - lo
                m = (rel >= 0) & (rel < PW)
                plsc.store_scatter(prio_v, [rel], pos, mask=m)
                return _

            lax.fori_loop(0, CH // 16, vec, None)
            return _

        lax.fori_loop(0, NCH, chunk, None)
        pltpu.sync_copy(prio_v, out_hbm.at[pl.ds(lo, PW)])

    return scatter_k(set_indices)


def _tc_argmax_full(queries, weight, prio, BK):
    NQ, D = queries.shape
    G = prio.shape[0] // BK

    def body(q_ref, k_ref, p_ref, o_ref, bval, bidx):
        i = pl.program_id(0)
        q = q_ref[...]
        k = k_ref[...]
        p = p_ref[...].reshape(1, BK)
        qn = q / jnp.maximum(
            jnp.sqrt(jnp.sum(q * q, axis=1, keepdims=True)), 1e-12)
        kn = k / jnp.maximum(
            jnp.sqrt(jnp.sum(k * k, axis=1, keepdims=True)), 1e-12)
        dn = (((1,), (1,)), ((), ()))
        s = lax.dot_general(qn.astype(jnp.bfloat16), kn.astype(jnp.bfloat16),
                            dn, preferred_element_type=jnp.float32)
        valid = p < _BIG
        s = jnp.where(valid, s, -jnp.inf)
        m = jnp.max(s, axis=1, keepdims=True)
        pc = jnp.min(jnp.where((s == m) & valid, p, _BIG), axis=1,
                     keepdims=True)

        @pl.when(i == 0)
        def _():
            bval[...] = jnp.full((NQ, 1), -jnp.inf, jnp.float32)
            bidx[...] = jnp.full((NQ, 1), _BIG, jnp.int32)

        old = bval[...]
        upd = m > old
        tie = m == old
        bval[...] = jnp.where(upd, m, old)
        bidx[...] = jnp.where(
            upd, pc, jnp.where(tie, jnp.minimum(bidx[...], pc), bidx[...]))

        @pl.when(i == G - 1)
        def _():
            o_ref[...] = bidx[...]

    return pl.pallas_call(
        body,
        grid=(G,),
        in_specs=[
            pl.BlockSpec((NQ, D), lambda i: (0, 0)),
            pl.BlockSpec((BK, D), lambda i: (i, 0)),
            pl.BlockSpec((BK,), lambda i: (i,)),
        ],
        out_specs=pl.BlockSpec((NQ, 1), lambda i: (0, 0)),
        out_shape=jax.ShapeDtypeStruct((NQ, 1), jnp.int32),
        scratch_shapes=[pltpu.VMEM((NQ, 1), jnp.float32),
                        pltpu.VMEM((NQ, 1), jnp.int32)],
    )(queries, weight, prio)


def kernel(embedded_inputs, embedding_weight, set_indices, topk):
    bsz, seq_len, emb_dim = embedded_inputs.shape
    queries = embedded_inputs.reshape(-1, emb_dim)
    V = embedding_weight.shape[0]
    BK = 4096
    G = pl.cdiv(V, BK)
    n_pad = G * BK
    prio = _sc_scatter_prio(set_indices, n_pad)
    pos = _tc_argmax_full(queries, embedding_weight, prio, BK).reshape(-1)
    full = jnp.take(set_indices, pos)
    emb = jnp.take(embedding_weight, full, axis=0)
    return emb.reshape(bsz, seq_len, emb_dim), full.reshape(bsz, seq_len)


# speedup vs baseline: 1.2053x; 1.2053x over previous
import functools

import jax
import jax.numpy as jnp
from jax import lax
from jax.experimental import pallas as pl
from jax.experimental.pallas import tpu as pltpu
from jax.experimental.pallas import tpu_sc as plsc

_NC = 2
_NS = 16
_NW = _NC * _NS

_BIG = 2**31 - 1


def _sc_mesh():
    return plsc.VectorSubcoreMesh(
        core_axis_name="c", subcore_axis_name="s", num_cores=_NC,
        num_subcores=_NS)


def _worker_id():
    return lax.axis_index("s") * _NC + lax.axis_index("c")


def _sc_scatter_prio(set_indices, n_pad):
    B = set_indices.shape[0]
    PW = n_pad // _NW
    CH = 10000
    NCH = B // CH

    @functools.partial(
        pl.kernel,
        out_type=jax.ShapeDtypeStruct((n_pad,), jnp.int32),
        mesh=_sc_mesh(),
        compiler_params=pltpu.CompilerParams(needs_layout_passes=False),
        scratch_types=[
            pltpu.VMEM((PW,), jnp.int32),
            pltpu.VMEM((CH,), jnp.int32),
        ],
    )
    def scatter_k(idx_hbm, out_hbm, prio_v, idx_v):
        wid = _worker_id()
        lo = wid * PW

        def init(i, _):
            prio_v[pl.ds(i * 16, 16)] = jnp.full((16,), _BIG, jnp.int32)
            return _

        lax.fori_loop(0, PW // 16, init, None)

        iota = lax.iota(jnp.int32, 16)

        def chunk(t, _):
            c = NCH - 1 - t
            pltpu.sync_copy(idx_hbm.at[pl.ds(c * CH, CH)], idx_v)

            def vec(j, _):
                jj = (CH // 16) - 1 - j
                v = idx_v[pl.ds(jj * 16, 16)]
                pos = (c * CH + jj * 16) + iota
                rel = v - lo
                m = (rel >= 0) & (rel < PW)
                plsc.store_scatter(prio_v, [rel], pos, mask=m)
                return _

            lax.fori_loop(0, CH // 16, vec, None)
            return _

        lax.fori_loop(0, NCH, chunk, None)
        pltpu.sync_copy(prio_v, out_hbm.at[pl.ds(lo, PW)])

    return scatter_k(set_indices)


def _tc_argmax_full(queries, weight, prio, BK):
    NQ, D = queries.shape
    G = prio.shape[0] // BK

    def body(q_ref, k_ref, p_ref, o_ref, bval, bidx):
        i = pl.program_id(0)
        q = q_ref[...]
        k = k_ref[...]
        p = p_ref[...].reshape(1, BK)
        qn = q / jnp.maximum(
            jnp.sqrt(jnp.sum(q * q, axis=1, keepdims=True)), 1e-12)
        kt = k.T
        kn = kt / jnp.maximum(
            jnp.sqrt(jnp.sum(kt * kt, axis=0, keepdims=True)), 1e-12)
        dn = (((1,), (0,)), ((), ()))
        s = lax.dot_general(qn.astype(jnp.bfloat16), kn.astype(jnp.bfloat16),
                            dn, preferred_element_type=jnp.float32)
        valid = p < _BIG
        s = jnp.where(valid, s, -jnp.inf)
        m = jnp.max(s, axis=1, keepdims=True)
        pc = jnp.min(jnp.where((s == m) & valid, p, _BIG), axis=1,
                     keepdims=True)

        @pl.when(i == 0)
        def _():
            bval[...] = jnp.full((NQ, 1), -jnp.inf, jnp.float32)
            bidx[...] = jnp.full((NQ, 1), _BIG, jnp.int32)

        old = bval[...]
        upd = m > old
        tie = m == old
        bval[...] = jnp.where(upd, m, old)
        bidx[...] = jnp.where(
            upd, pc, jnp.where(tie, jnp.minimum(bidx[...], pc), bidx[...]))

        @pl.when(i == G - 1)
        def _():
            o_ref[...] = bidx[...]

    return pl.pallas_call(
        body,
        grid=(G,),
        in_specs=[
            pl.BlockSpec((NQ, D), lambda i: (0, 0)),
            pl.BlockSpec((BK, D), lambda i: (i, 0)),
            pl.BlockSpec((BK,), lambda i: (i,)),
        ],
        out_specs=pl.BlockSpec((NQ, 1), lambda i: (0, 0)),
        out_shape=jax.ShapeDtypeStruct((NQ, 1), jnp.int32),
        scratch_shapes=[pltpu.VMEM((NQ, 1), jnp.float32),
                        pltpu.VMEM((NQ, 1), jnp.int32)],
    )(queries, weight, prio)


def kernel(embedded_inputs, embedding_weight, set_indices, topk):
    bsz, seq_len, emb_dim = embedded_inputs.shape
    queries = embedded_inputs.reshape(-1, emb_dim)
    V = embedding_weight.shape[0]
    BK = 8192
    G = pl.cdiv(V, BK)
    n_pad = G * BK
    prio = _sc_scatter_prio(set_indices, n_pad)
    pos = _tc_argmax_full(queries, embedding_weight, prio, BK).reshape(-1)
    full = jnp.take(set_indices, pos)
    emb = jnp.take(embedding_weight, full, axis=0)
    return emb.reshape(bsz, seq_len, emb_dim), full.reshape(bsz, seq_len)


# speedup vs baseline: 1.3009x; 1.0793x over previous
import functools

import jax
import jax.numpy as jnp
from jax import lax
from jax.experimental import pallas as pl
from jax.experimental.pallas import tpu as pltpu
from jax.experimental.pallas import tpu_sc as plsc

_NC = 2
_NS = 16
_NW = _NC * _NS


def _sc_mesh():
    return plsc.VectorSubcoreMesh(
        core_axis_name="c", subcore_axis_name="s", num_cores=_NC,
        num_subcores=_NS)


def _worker_id():
    return lax.axis_index("s") * _NC + lax.axis_index("c")


def _sc_gather(weight, set_indices):
    B = set_indices.shape[0]
    D = weight.shape[1]
    CH = 512
    K = 16
    NCH = pl.cdiv(B, CH)
    SLOTS = pl.cdiv(NCH, _NW)

    @functools.partial(
        pl.kernel,
        out_type=jax.ShapeDtypeStruct((B, D), jnp.float32),
        mesh=_sc_mesh(),
        compiler_params=pltpu.CompilerParams(use_tc_tiling_on_sc=True),
        scratch_types=[
            pltpu.VMEM((CH,), jnp.int32),
            pltpu.VMEM((CH, D), jnp.float32),
            pltpu.SemaphoreType.DMA,
        ],
    )
    def gather_k(w_hbm, idx_hbm, out_hbm, idx_v, rows_v, gsem):
        wid = _worker_id()
        for slot in range(SLOTS):
            c = wid + slot * _NW

            @pl.when(c < NCH)
            def _():
                start = jnp.minimum(c * CH, B - CH)
                start = pl.multiple_of(start, 8)
                pltpu.sync_copy(idx_hbm.at[pl.ds(start, CH)], idx_v)

                def burst(i, _):
                    vec = idx_v[pl.ds(i * K, K)]
                    cps = []
                    for j in range(K):
                        row = vec[j]
                        cps.append(pltpu.async_copy(
                            w_hbm.at[pl.ds(row, 1)],
                            rows_v.at[pl.ds(i * K + j, 1)],
                            gsem))
                    for cp in cps:
                        cp.wait()
                    return _

                lax.fori_loop(0, CH // K, burst, None)
                pltpu.sync_copy(rows_v, out_hbm.at[pl.ds(start, CH)])

    return gather_k(weight, set_indices)


def _tc_argmax(queries, keys, BK):
    NQ, D = queries.shape
    B = keys.shape[0]
    G = pl.cdiv(B, BK)

    def body(q_ref, k_ref, o_ref, bval, bidx):
        i = pl.program_id(0)
        q = q_ref[...]
        k = k_ref[...]
        qn = q / jnp.maximum(
            jnp.sqrt(jnp.sum(q * q, axis=1, keepdims=True)), 1e-12)
        kt = k.T
        kn = kt / jnp.maximum(
            jnp.sqrt(jnp.sum(kt * kt, axis=0, keepdims=True)), 1e-12)
        dn = (((1,), (0,)), ((), ()))
        s = lax.dot_general(qn.astype(jnp.bfloat16), kn.astype(jnp.bfloat16),
                            dn, preferred_element_type=jnp.float32)
        gid = i * BK + lax.broadcasted_iota(jnp.int32, (NQ, BK), 1)
        s = jnp.where(gid < B, s, -jnp.inf)
        m = jnp.max(s, axis=1, keepdims=True)
        cidx = jnp.min(jnp.where(s == m, gid, jnp.int32(B)), axis=1,
                       keepdims=True)

        @pl.when(i == 0)
        def _():
            bval[...] = jnp.full((NQ, 1), -jnp.inf, jnp.float32)
            bidx[...] = jnp.zeros((NQ, 1), jnp.int32)

        upd = m > bval[...]
        bval[...] = jnp.where(upd, m, bval[...])
        bidx[...] = jnp.where(upd, cidx, bidx[...])

        @pl.when(i == G - 1)
        def _():
            o_ref[...] = bidx[...]

    return pl.pallas_call(
        body,
        grid=(G,),
        in_specs=[
            pl.BlockSpec((NQ, D), lambda i: (0, 0)),
            pl.BlockSpec((BK, D), lambda i: (i, 0)),
        ],
        out_specs=pl.BlockSpec((NQ, 1), lambda i: (0, 0)),
        out_shape=jax.ShapeDtypeStruct((NQ, 1), jnp.int32),
        scratch_shapes=[pltpu.VMEM((NQ, 1), jnp.float32),
                        pltpu.VMEM((NQ, 1), jnp.int32)],
    )(queries, keys)


def kernel(embedded_inputs, embedding_weight, set_indices, topk):
    bsz, seq_len, emb_dim = embedded_inputs.shape
    queries = embedded_inputs.reshape(-1, emb_dim)
    keys = _sc_gather(embedding_weight, set_indices)
    argidx = _tc_argmax(queries, keys, 8192).reshape(-1)
    full = jnp.take(set_indices, argidx)
    emb = jnp.take(embedding_weight, full, axis=0)
    return emb.reshape(bsz, seq_len, emb_dim), full.reshape(bsz, seq_len)


# speedup vs baseline: 1.5019x; 1.1545x over previous
import functools

import jax
import jax.numpy as jnp
from jax import lax
from jax.experimental import pallas as pl
from jax.experimental.pallas import tpu as pltpu
from jax.experimental.pallas import tpu_sc as plsc

_NC = 2
_NS = 16
_NW = _NC * _NS


def _sc_mesh():
    return plsc.VectorSubcoreMesh(
        core_axis_name="c", subcore_axis_name="s", num_cores=_NC,
        num_subcores=_NS)


def _worker_id():
    return lax.axis_index("s") * _NC + lax.axis_index("c")


def _sc_gather(weight, set_indices):
    B = set_indices.shape[0]
    D = weight.shape[1]
    CH = 512
    K = 16
    NCH = pl.cdiv(B, CH)
    SLOTS = pl.cdiv(NCH, _NW)

    @functools.partial(
        pl.kernel,
        out_type=jax.ShapeDtypeStruct((B, D), jnp.float32),
        mesh=_sc_mesh(),
        compiler_params=pltpu.CompilerParams(use_tc_tiling_on_sc=True),
        scratch_types=[
            pltpu.VMEM((CH,), jnp.int32),
            pltpu.VMEM((CH, D), jnp.float32),
            pltpu.SemaphoreType.DMA,
        ],
    )
    def gather_k(w_hbm, idx_hbm, out_hbm, idx_v, rows_v, gsem):
        wid = _worker_id()
        for slot in range(SLOTS):
            c = wid + slot * _NW

            @pl.when(c < NCH)
            def _():
                start = jnp.minimum(c * CH, B - CH)
                start = pl.multiple_of(start, 8)
                pltpu.sync_copy(idx_hbm.at[pl.ds(start, CH)], idx_v)

                def drain_one():
                    pltpu.make_async_copy(
                        w_hbm.at[pl.ds(0, K)], rows_v.at[pl.ds(0, K)],
                        gsem).wait()

                def burst(i, _):
                    vec = idx_v[pl.ds(i * K, K)]
                    for j in range(K):
                        row = vec[j]
                        pltpu.async_copy(
                            w_hbm.at[pl.ds(row, 1)],
                            rows_v.at[pl.ds(i * K + j, 1)],
                            gsem)

                    @pl.when(i > 0)
                    def _():
                        drain_one()

                    return _

                lax.fori_loop(0, CH // K, burst, None)
                drain_one()
                pltpu.sync_copy(rows_v, out_hbm.at[pl.ds(start, CH)])

    return gather_k(weight, set_indices)


def _tc_argmax(queries, keys, BK):
    NQ, D = queries.shape
    B = keys.shape[0]
    G = pl.cdiv(B, BK)

    def body(q_ref, k_ref, o_ref, bval, bidx):
        i = pl.program_id(0)
        q = q_ref[...]
        k = k_ref[...]
        qn = q / jnp.maximum(
            jnp.sqrt(jnp.sum(q * q, axis=1, keepdims=True)), 1e-12)
        kt = k.T
        kn = kt / jnp.maximum(
            jnp.sqrt(jnp.sum(kt * kt, axis=0, keepdims=True)), 1e-12)
        dn = (((1,), (0,)), ((), ()))
        s = lax.dot_general(qn.astype(jnp.bfloat16), kn.astype(jnp.bfloat16),
                            dn, preferred_element_type=jnp.float32)
        gid = i * BK + lax.broadcasted_iota(jnp.int32, (NQ, BK), 1)
        s = jnp.where(gid < B, s, -jnp.inf)
        m = jnp.max(s, axis=1, keepdims=True)
        cidx = jnp.min(jnp.where(s == m, gid, jnp.int32(B)), axis=1,
                       keepdims=True)

        @pl.when(i == 0)
        def _():
            bval[...] = jnp.full((NQ, 1), -jnp.inf, jnp.float32)
            bidx[...] = jnp.zeros((NQ, 1), jnp.int32)

        upd = m > bval[...]
        bval[...] = jnp.where(upd, m, bval[...])
        bidx[...] = jnp.where(upd, cidx, bidx[...])

        @pl.when(i == G - 1)
        def _():
            o_ref[...] = bidx[...]

    return pl.pallas_call(
        body,
        grid=(G,),
        in_specs=[
            pl.BlockSpec((NQ, D), lambda i: (0, 0)),
            pl.BlockSpec((BK, D), lambda i: (i, 0)),
        ],
        out_specs=pl.BlockSpec((NQ, 1), lambda i: (0, 0)),
        out_shape=jax.ShapeDtypeStruct((NQ, 1), jnp.int32),
        scratch_shapes=[pltpu.VMEM((NQ, 1), jnp.float32),
                        pltpu.VMEM((NQ, 1), jnp.int32)],
    )(queries, keys)


def kernel(embedded_inputs, embedding_weight, set_indices, topk):
    bsz, seq_len, emb_dim = embedded_inputs.shape
    queries = embedded_inputs.reshape(-1, emb_dim)
    keys = _sc_gather(embedding_weight, set_indices)
    argidx = _tc_argmax(queries, keys, 8192).reshape(-1)
    full = jnp.take(set_indices, argidx)
    emb = jnp.take(embedding_weight, full, axis=0)
    return emb.reshape(bsz, seq_len, emb_dim), full.reshape(bsz, seq_len)


# speedup vs baseline: 1.5830x; 1.0540x over previous
import functools

import jax
import jax.numpy as jnp
from jax import lax
from jax.experimental import pallas as pl
from jax.experimental.pallas import tpu as pltpu
from jax.experimental.pallas import tpu_sc as plsc

_NC = 2
_NS = 16
_NW = _NC * _NS


def _sc_mesh():
    return plsc.VectorSubcoreMesh(
        core_axis_name="c", subcore_axis_name="s", num_cores=_NC,
        num_subcores=_NS)


def _worker_id():
    return lax.axis_index("s") * _NC + lax.axis_index("c")


def _sc_gather(weight, set_indices):
    B = set_indices.shape[0]
    D = weight.shape[1]
    CH = 512
    K = 16
    NCH = pl.cdiv(B, CH)
    SLOTS = pl.cdiv(NCH, _NW)

    @functools.partial(
        pl.kernel,
        out_type=jax.ShapeDtypeStruct((B, D), jnp.float32),
        mesh=_sc_mesh(),
        compiler_params=pltpu.CompilerParams(use_tc_tiling_on_sc=True),
        scratch_types=[
            pltpu.VMEM((CH,), jnp.int32),
            pltpu.VMEM((CH, D), jnp.float32),
            pltpu.SemaphoreType.DMA,
        ],
    )
    def gather_k(w_hbm, idx_hbm, out_hbm, idx_v, rows_v, gsem):
        wid = _worker_id()
        for slot in range(SLOTS):
            c = wid + slot * _NW

            @pl.when(c < NCH)
            def _():
                start = jnp.minimum(c * CH, B - CH)
                start = pl.multiple_of(start, 8)
                pltpu.sync_copy(idx_hbm.at[pl.ds(start, CH)], idx_v)

                def drain_one():
                    pltpu.make_async_copy(
                        w_hbm.at[pl.ds(0, K)], rows_v.at[pl.ds(0, K)],
                        gsem).wait()

                def burst(i, _):
                    vec = idx_v[pl.ds(i * K, K)]
                    for j in range(K):
                        row = vec[j]
                        pltpu.async_copy(
                            w_hbm.at[pl.ds(row, 1)],
                            rows_v.at[pl.ds(i * K + j, 1)],
                            gsem)

                    @pl.when(i > 1)
                    def _():
                        drain_one()

                    return _

                lax.fori_loop(0, CH // K, burst, None)
                drain_one()
                drain_one()
                pltpu.sync_copy(rows_v, out_hbm.at[pl.ds(start, CH)])

    return gather_k(weight, set_indices)


def _tc_argmax(queries, keys, BK):
    NQ, D = queries.shape
    B = keys.shape[0]
    G = pl.cdiv(B, BK)

    def body(q_ref, k_ref, o_ref, bval, bidx):
        i = pl.program_id(0)
        q = q_ref[...]
        k = k_ref[...]
        qn = q / jnp.maximum(
            jnp.sqrt(jnp.sum(q * q, axis=1, keepdims=True)), 1e-12)
        kt = k.T
        kn = kt / jnp.maximum(
            jnp.sqrt(jnp.sum(kt * kt, axis=0, keepdims=True)), 1e-12)
        dn = (((1,), (0,)), ((), ()))
        s = lax.dot_general(qn.astype(jnp.bfloat16), kn.astype(jnp.bfloat16),
                            dn, preferred_element_type=jnp.float32)
        gid = i * BK + lax.broadcasted_iota(jnp.int32, (NQ, BK), 1)
        s = jnp.where(gid < B, s, -jnp.inf)
        m = jnp.max(s, axis=1, keepdims=True)
        cidx = jnp.min(jnp.where(s == m, gid, jnp.int32(B)), axis=1,
                       keepdims=True)

        @pl.when(i == 0)
        def _():
            bval[...] = jnp.full((NQ, 1), -jnp.inf, jnp.float32)
            bidx[...] = jnp.zeros((NQ, 1), jnp.int32)

        upd = m > bval[...]
        bval[...] = jnp.where(upd, m, bval[...])
        bidx[...] = jnp.where(upd, cidx, bidx[...])

        @pl.when(i == G - 1)
        def _():
            o_ref[...] = bidx[...]

    return pl.pallas_call(
        body,
        grid=(G,),
        in_specs=[
            pl.BlockSpec((NQ, D), lambda i: (0, 0)),
            pl.BlockSpec((BK, D), lambda i: (i, 0)),
        ],
        out_specs=pl.BlockSpec((NQ, 1), lambda i: (0, 0)),
        out_shape=jax.ShapeDtypeStruct((NQ, 1), jnp.int32),
        scratch_shapes=[pltpu.VMEM((NQ, 1), jnp.float32),
                        pltpu.VMEM((NQ, 1), jnp.int32)],
    )(queries, keys)


def kernel(embedded_inputs, embedding_weight, set_indices, topk):
    bsz, seq_len, emb_dim = embedded_inputs.shape
    queries = embedded_inputs.reshape(-1, emb_dim)
    keys = _sc_gather(embedding_weight, set_indices)
    argidx = _tc_argmax(queries, keys, 8192).reshape(-1)
    full = jnp.take(set_indices, argidx)
    emb = jnp.take(embedding_weight, full, axis=0)
    return emb.reshape(bsz, seq_len, emb_dim), full.reshape(bsz, seq_len)


# speedup vs baseline: 1.6476x; 1.0409x over previous
import functools

import jax
import jax.numpy as jnp
from jax import lax
from jax.experimental import pallas as pl
from jax.experimental.pallas import tpu as pltpu
from jax.experimental.pallas import tpu_sc as plsc

_NC = 2
_NS = 16
_NW = _NC * _NS


def _sc_mesh():
    return plsc.VectorSubcoreMesh(
        core_axis_name="c", subcore_axis_name="s", num_cores=_NC,
        num_subcores=_NS)


def _worker_id():
    return lax.axis_index("s") * _NC + lax.axis_index("c")


def _sc_gather(weight, set_indices):
    B = set_indices.shape[0]
    D = weight.shape[1]
    CH = 512
    K = 16
    NCH = pl.cdiv(B, CH)
    SLOTS = pl.cdiv(NCH, _NW)

    @functools.partial(
        pl.kernel,
        out_type=jax.ShapeDtypeStruct((B, D), jnp.float32),
        mesh=_sc_mesh(),
        compiler_params=pltpu.CompilerParams(use_tc_tiling_on_sc=True),
        scratch_types=[
            pltpu.VMEM((CH,), jnp.int32),
            pltpu.VMEM((CH, D), jnp.float32),
            pltpu.SemaphoreType.DMA,
        ],
    )
    def gather_k(w_hbm, idx_hbm, out_hbm, idx_v, rows_v, gsem):
        wid = _worker_id()
        for slot in range(SLOTS):
            c = wid + slot * _NW

            @pl.when(c < NCH)
            def _():
                start = jnp.minimum(c * CH, B - CH)
                start = pl.multiple_of(start, 8)
                pltpu.sync_copy(idx_hbm.at[pl.ds(start, CH)], idx_v)

                def drain_one():
                    pltpu.make_async_copy(
                        w_hbm.at[pl.ds(0, K)], rows_v.at[pl.ds(0, K)],
                        gsem).wait()

                def burst(i, _):
                    vec = idx_v[pl.ds(i * K, K)]
                    for j in range(K):
                        row = vec[j]
                        pltpu.async_copy(
                            w_hbm.at[pl.ds(row, 1)],
                            rows_v.at[pl.ds(i * K + j, 1)],
                            gsem)

                    @pl.when(i > 3)
                    def _():
                        drain_one()

                    return _

                lax.fori_loop(0, CH // K, burst, None)
                for _ in range(4):
                    drain_one()
                pltpu.sync_copy(rows_v, out_hbm.at[pl.ds(start, CH)])

    return gather_k(weight, set_indices)


def _tc_argmax(queries, keys, BK):
    NQ, D = queries.shape
    B = keys.shape[0]
    G = pl.cdiv(B, BK)

    def body(q_ref, k_ref, o_ref, bval, bidx):
        i = pl.program_id(0)
        q = q_ref[...]
        k = k_ref[...]
        qn = q / jnp.maximum(
            jnp.sqrt(jnp.sum(q * q, axis=1, keepdims=True)), 1e-12)
        kt = k.T
        kn = kt / jnp.maximum(
            jnp.sqrt(jnp.sum(kt * kt, axis=0, keepdims=True)), 1e-12)
        dn = (((1,), (0,)), ((), ()))
        s = lax.dot_general(qn.astype(jnp.bfloat16), kn.astype(jnp.bfloat16),
                            dn, preferred_element_type=jnp.float32)
        gid = i * BK + lax.broadcasted_iota(jnp.int32, (NQ, BK), 1)
        s = jnp.where(gid < B, s, -jnp.inf)
        m = jnp.max(s, axis=1, keepdims=True)
        cidx = jnp.min(jnp.where(s == m, gid, jnp.int32(B)), axis=1,
                       keepdims=True)

        @pl.when(i == 0)
        def _():
            bval[...] = jnp.full((NQ, 1), -jnp.inf, jnp.float32)
            bidx[...] = jnp.zeros((NQ, 1), jnp.int32)

        upd = m > bval[...]
        bval[...] = jnp.where(upd, m, bval[...])
        bidx[...] = jnp.where(upd, cidx, bidx[...])

        @pl.when(i == G - 1)
        def _():
            o_ref[...] = bidx[...]

    return pl.pallas_call(
        body,
        grid=(G,),
        in_specs=[
            pl.BlockSpec((NQ, D), lambda i: (0, 0)),
            pl.BlockSpec((BK, D), lambda i: (i, 0)),
        ],
        out_specs=pl.BlockSpec((NQ, 1), lambda i: (0, 0)),
        out_shape=jax.ShapeDtypeStruct((NQ, 1), jnp.int32),
        scratch_shapes=[pltpu.VMEM((NQ, 1), jnp.float32),
                        pltpu.VMEM((NQ, 1), jnp.int32)],
    )(queries, keys)


def kernel(embedded_inputs, embedding_weight, set_indices, topk):
    bsz, seq_len, emb_dim = embedded_inputs.shape
    queries = embedded_inputs.reshape(-1, emb_dim)
    keys = _sc_gather(embedding_weight, set_indices)
    argidx = _tc_argmax(queries, keys, 8192).reshape(-1)
    full = jnp.take(set_indices, argidx)
    emb = jnp.take(embedding_weight, full, axis=0)
    return emb.reshape(bsz, seq_len, emb_dim), full.reshape(bsz, seq_len)
